# baseline (speedup 1.0000x reference)
import jax
import jax.numpy as jnp
from jax import lax
from jax.experimental import pallas as pl
from jax.experimental.pallas import tpu as pltpu

VOCAB = 100000
EMBED_DIM = 64
CONTEXT = 50
HIDDEN = 128

BV = 8192
NB = (VOCAB + BV - 1) // BV
VPAD = NB * BV

_NEG = -1e30


_WSPLIT = 4
_WROWS = HIDDEN // _WSPLIT


def _body(idx_ref, *refs):
    row_refs = refs[:CONTEXT]
    (w1_ref, b1_ref, *w2_refs, b2_ref, o_ref, h_ref, ms_ref) = refs[CONTEXT:]
    j = pl.program_id(0)

    @pl.when(j == 0)
    def _():
        h = b1_ref[...]
        sub = lax.broadcasted_iota(jnp.int32, (8, 1), 0)
        for k in range(CONTEXT):
            slab = row_refs[k][...]
            row = jnp.sum(jnp.where(sub == idx_ref[k] % 8, slab, 0.0),
                          axis=0, keepdims=True)
            h = h + jnp.dot(row,
                            w1_ref[pl.ds(k * EMBED_DIM, EMBED_DIM), :],
                            preferred_element_type=jnp.float32)
        h_ref[...] = jnp.maximum(h, 0.0)
        ms_ref[0] = _NEG
        ms_ref[1] = 0.0

    logits = b2_ref[...]
    for q in range(_WSPLIT):
        logits = logits + jnp.dot(
            h_ref[:, pl.ds(q * _WROWS, _WROWS)], w2_refs[q][...],
            preferred_element_type=jnp.float32)
    col = j * BV + lax.broadcasted_iota(jnp.int32, (1, BV), 1)
    logits = jnp.where(col < VOCAB, logits, _NEG)
    o_ref[:, pl.ds(j * BV, BV)] = logits

    m_old = ms_ref[0]
    s_old = ms_ref[1]
    m_new = jnp.maximum(m_old, jnp.max(logits))
    s_new = s_old * jnp.exp(m_old - m_new) + jnp.sum(jnp.exp(logits - m_new))
    ms_ref[0] = m_new
    ms_ref[1] = s_new

    @pl.when(j == NB - 1)
    def _():
        o_ref[...] = o_ref[...] - (m_new + jnp.log(s_new))


def _row_spec(k):
    return pl.BlockSpec((8, EMBED_DIM), lambda j, idx, _k=k: (idx[_k] // 8, 0))


_grid_spec = pltpu.PrefetchScalarGridSpec(
    num_scalar_prefetch=1,
    grid=(NB,),
    in_specs=[
        *[_row_spec(k) for k in range(CONTEXT)],
        pl.BlockSpec((CONTEXT * EMBED_DIM, HIDDEN), lambda j, idx: (0, 0)),
        pl.BlockSpec((1, HIDDEN), lambda j, idx: (0, 0)),
        *[pl.BlockSpec((_WROWS, BV), lambda j, idx, _q=q: (_q, j))
          for q in range(_WSPLIT)],
        pl.BlockSpec((1, BV), lambda j, idx: (0, j)),
    ],
    out_specs=pl.BlockSpec((1, VPAD), lambda j, idx: (0, 0)),
    scratch_shapes=[
        pltpu.VMEM((1, HIDDEN), jnp.float32),
        pltpu.SMEM((2,), jnp.float32),
    ],
)

_mlp_call = pl.pallas_call(
    _body,
    grid_spec=_grid_spec,
    out_shape=jax.ShapeDtypeStruct((1, VPAD), jnp.float32),
)


def kernel(inputs, table, W1, b1, W2, b2):
    idx = inputs.astype(jnp.int32)
    out = _mlp_call(idx, *([table] * CONTEXT), W1, b1.reshape(1, HIDDEN),
                    *([W2] * _WSPLIT), b2.reshape(1, VOCAB))
    return out[:, :VOCAB]
